# Initial kernel scaffold; baseline (speedup 1.0000x reference)
import jax
import jax.numpy as jnp
from jax.experimental import pallas as pl


def kernel(inputs, table):
    raise NotImplementedError("write your pallas kernel here")


# speedup vs baseline: 1.3661x; 1.3661x over previous
import functools

import jax
import jax.numpy as jnp
from jax import lax
from jax.experimental import pallas as pl
from jax.experimental.pallas import tpu as pltpu
from jax.experimental.pallas import tpu_sc as plsc

EMB = 32
BATCH = 16384
FIELDS = 26
B_TOTAL = BATCH * FIELDS
NC = 2
NS = 16
NW = NC * NS
B_PER_W = B_TOTAL // NW
CHUNK = 128
NCHUNK = B_PER_W // CHUNK


def _body(table_hbm, idx_hbm, out_hbm, idx_c, rows, sem):
    wid = lax.axis_index("s") * NC + lax.axis_index("c")
    base = wid * B_PER_W

    def step(i, carry):
        off = pl.multiple_of(base + i * CHUNK, 8)
        pltpu.sync_copy(idx_hbm.at[pl.ds(off, CHUNK)], idx_c)
        pltpu.async_copy(table_hbm.at[idx_c], rows, sem).wait()
        pltpu.sync_copy(rows, out_hbm.at[pl.ds(off, CHUNK)])
        return carry

    lax.fori_loop(0, NCHUNK, step, 0)


@jax.jit
def _lookup(table, idx):
    mesh = plsc.VectorSubcoreMesh(core_axis_name="c", subcore_axis_name="s")
    return pl.kernel(
        _body,
        out_type=jax.ShapeDtypeStruct((B_TOTAL, EMB), jnp.float32),
        mesh=mesh,
        scratch_types=[
            pltpu.VMEM((CHUNK,), jnp.int32),
            pltpu.VMEM((CHUNK, EMB), jnp.float32),
            pltpu.SemaphoreType.DMA,
        ],
        compiler_params=pltpu.CompilerParams(use_tc_tiling_on_sc=False),
    )(table, idx)


def kernel(inputs, table):
    idx = inputs.reshape(-1).astype(jnp.int32)
    flat = _lookup(table, idx)
    return flat.reshape(BATCH, FIELDS, EMB)


# speedup vs baseline: 1.5743x; 1.1524x over previous
import jax
import jax.numpy as jnp
from jax import lax
from jax.experimental import pallas as pl
from jax.experimental.pallas import tpu as pltpu
from jax.experimental.pallas import tpu_sc as plsc

EMB = 32
BATCH = 16384
FIELDS = 26
B_TOTAL = BATCH * FIELDS
NC = 2
NS = 16
NW = NC * NS
B_PER_W = B_TOTAL // NW
CHUNK = 1024
NCHUNK = B_PER_W // CHUNK


def _body(table_hbm, idx_hbm, out_hbm, idx_v, rows, gsem, ssem):
    wid = lax.axis_index("s") * NC + lax.axis_index("c")
    base = wid * B_PER_W
    pltpu.sync_copy(idx_hbm.at[pl.ds(base, B_PER_W)], idx_v)

    def gather(i, p):
        return pltpu.make_async_copy(
            table_hbm.at[idx_v.at[pl.ds(i * CHUNK, CHUNK)]], rows.at[p], gsem[p]
        )

    def store(i, p):
        return pltpu.make_async_copy(
            rows.at[p], out_hbm.at[pl.ds(base + i * CHUNK, CHUNK)], ssem[p]
        )

    gather(0, 0).start()
    for i in range(NCHUNK):
        p = i % 2
        if i + 1 < NCHUNK:
            if i >= 1:
                store(i - 1, 1 - p).wait()
            gather(i + 1, 1 - p).start()
        gather(i, p).wait()
        store(i, p).start()
    store(NCHUNK - 2, NCHUNK % 2).wait()
    store(NCHUNK - 1, (NCHUNK - 1) % 2).wait()


@jax.jit
def _lookup(table, idx):
    mesh = plsc.VectorSubcoreMesh(core_axis_name="c", subcore_axis_name="s")
    return pl.kernel(
        _body,
        out_type=jax.ShapeDtypeStruct((B_TOTAL, EMB), jnp.float32),
        mesh=mesh,
        scratch_types=[
            pltpu.VMEM((B_PER_W,), jnp.int32),
            pltpu.VMEM((2, CHUNK, EMB), jnp.float32),
            (pltpu.SemaphoreType.DMA, pltpu.SemaphoreType.DMA),
            (pltpu.SemaphoreType.DMA, pltpu.SemaphoreType.DMA),
        ],
        compiler_params=pltpu.CompilerParams(use_tc_tiling_on_sc=False),
    )(table, idx)


def kernel(inputs, table):
    idx = inputs.reshape(-1).astype(jnp.int32)
    flat = _lookup(table, idx)
    return flat.reshape(BATCH, FIELDS, EMB)
